# baseline (speedup 1.0000x reference)
import functools

import jax
import jax.numpy as jnp
from jax import lax
from jax.experimental import pallas as pl
from jax.experimental.pallas import tpu as pltpu
from jax.experimental.pallas import tpu_sc as plsc

_NC = 2
_NS = 16
_NW = _NC * _NS

_NB = 8
_NH = 2
_L = 16


def _emb(token_ids, Ec, batch, hist, d):
    ch = _NB * hist
    hh = _NH * hist
    b_per_w = batch // _NW
    n_ch = b_per_w // _NB

    mesh = plsc.VectorSubcoreMesh(core_axis_name="c", subcore_axis_name="s")

    offs = []
    o = 0
    while o + _L < hist:
        offs.append(o)
        o += _L
    offs.append(hist - _L)

    @functools.partial(
        pl.kernel,
        mesh=mesh,
        out_type=jax.ShapeDtypeStruct((batch, hist, d), jnp.float32),
        compiler_params=pltpu.CompilerParams(use_tc_tiling_on_sc=True),
        scratch_types=[
            pltpu.VMEM((_NB, hist), jnp.int32),
            pltpu.VMEM((_NB, hist), jnp.int32),
            pltpu.VMEM((4, hh), jnp.int32),
            pltpu.VMEM((4, hh), jnp.int32),
            pltpu.VMEM((ch + _L,), jnp.int32),
            pltpu.VMEM((ch + _L,), jnp.int32),
            pltpu.VMEM((hh, 4 * d), jnp.float32),
            pltpu.VMEM((hh, 4 * d), jnp.float32),
            pltpu.VMEM((hh, 4 * d), jnp.float32),
            pltpu.VMEM((hh, 4 * d), jnp.float32),
            pltpu.VMEM((_NB, hist, d), jnp.float32),
            pltpu.SemaphoreType.DMA,
            pltpu.SemaphoreType.DMA,
            pltpu.SemaphoreType.DMA,
            pltpu.SemaphoreType.DMA,
            pltpu.SemaphoreType.DMA,
            pltpu.SemaphoreType.DMA,
        ],
    )
    def emb_kernel(idx_hbm, table_hbm, out_hbm,
                   idx0, idx1, q0, q1, sel0, sel1,
                   gbuf0, gbuf1, gbuf2, gbuf3, obuf,
                   semI, semG0, semG1, semG2, semG3, semO):
        gbufs = (gbuf0, gbuf1, gbuf2, gbuf3)
        semGs = (semG0, semG1, semG2, semG3)
        wid = lax.axis_index("s") * _NC + lax.axis_index("c")
        w_b0 = wid * b_per_w

        def build_q(idxbuf, q_ref, sel_v):
            for r in range(_NB):
                qt = r // _NH
                lbase = (r % _NH) * hist
                for off in offs:
                    x = idxbuf[r, pl.ds(off, _L)]
                    q_ref[qt, pl.ds(lbase + off, _L)] = x >> 2
                    sel_v[pl.ds(r * hist + off, _L)] = (x & 3) << 5

        def select_half(rlo, gval, sel_v):
            def r_body(r, _):
                ibase = r * hist
                gbase = (r - rlo) * hist
                for h0 in offs:
                    sv = sel_v[pl.ds(ibase + h0, _L)]
                    for k in range(_L):
                        cb = sv[k]
                        gi = gbase + h0 + k
                        obuf[r, h0 + k, pl.ds(0, _L)] = \
                            gval[gi, pl.ds(cb, _L)]
                        obuf[r, h0 + k, pl.ds(_L, _L)] = \
                            gval[gi, pl.ds(cb + _L, _L)]
                return ()
            lax.fori_loop(rlo, rlo + _NH, r_body, ())

        def phase(c, idx_cur, idx_nxt, q_cur, q_nxt, sel_cur, sel_nxt):
            b0 = w_b0 + c * _NB

            @pl.when(c + 1 < n_ch)
            def _():
                pltpu.make_async_copy(
                    idx_hbm.at[pl.ds(b0 + _NB, _NB)], idx_nxt, semI).wait()
                build_q(idx_nxt, q_nxt, sel_nxt)

            @pl.when(c > 0)
            def _():
                pltpu.make_async_copy(
                    obuf, out_hbm.at[pl.ds(b0, _NB)], semO).wait()

            for qt in range(4):
                pltpu.make_async_copy(
                    table_hbm.at[q_cur.at[qt]], gbufs[qt], semGs[qt]).wait()
                select_half(qt * _NH, gbufs[qt], sel_cur)

                @pl.when(c + 1 < n_ch)
                def _():
                    pltpu.async_copy(
                        table_hbm.at[q_nxt.at[qt]], gbufs[qt], semGs[qt])

            pltpu.async_copy(obuf, out_hbm.at[pl.ds(b0, _NB)], semO)

            @pl.when(c + 2 < n_ch)
            def _():
                pltpu.async_copy(
                    idx_hbm.at[pl.ds(b0 + 2 * _NB, _NB)], idx_cur, semI)

        pltpu.sync_copy(idx_hbm.at[pl.ds(w_b0, _NB)], idx0)
        build_q(idx0, q0, sel0)
        for qt in range(4):
            pltpu.async_copy(table_hbm.at[q0.at[qt]], gbufs[qt], semGs[qt])
        pltpu.async_copy(idx_hbm.at[pl.ds(w_b0 + _NB, _NB)], idx1, semI)

        def pair_body(k, _):
            c = 2 * k
            phase(c, idx0, idx1, q0, q1, sel0, sel1)
            phase(c + 1, idx1, idx0, q1, q0, sel1, sel0)
            return ()

        lax.fori_loop(0, n_ch // 2, pair_body, ())
        pltpu.make_async_copy(
            obuf,
            out_hbm.at[pl.ds(w_b0 + (n_ch - 1) * _NB, _NB)],
            semO).wait()

    return emb_kernel(token_ids, Ec)


def kernel(token_ids, E):
    batch, hist = token_ids.shape
    v, d = E.shape
    Ec = E.reshape(v // 4, 4 * d)
    return _emb(token_ids.astype(jnp.int32), Ec, batch, hist, d)
